# baseline (speedup 1.0000x reference)
import jax
import jax.numpy as jnp
from jax import lax
from jax.experimental import pallas as pl
from jax.experimental.pallas import tpu as pltpu
from jax.experimental.pallas import tpu_sc as plsc

BATCH = 16384
N_FIELDS = 26
N_EMB = 13
VOCAB = 500
EMB_DIM = 4
TAB_SZ = VOCAB * EMB_DIM
OUT_DIM = N_EMB * EMB_DIM + (N_FIELDS - 1 - N_EMB)
NUM_CORES = 2
NUM_SUBCORES = 16
NUM_WORKERS = NUM_CORES * NUM_SUBCORES
ROWS_PER_WORKER = BATCH // NUM_WORKERS
HALF = ROWS_PER_WORKER // 2
LANES = 16
OUT_MINOR = 128
OUT_MAJOR = BATCH * OUT_DIM // OUT_MINOR
OROWS_PER_WORKER = ROWS_PER_WORKER * OUT_DIM // OUT_MINOR


def _sc_body(user_sparse_hbm, tables_hbm, out_hbm, in_v, tab_v, out_v):
    wid = lax.axis_index("s") * NUM_CORES + lax.axis_index("c")
    base = wid * ROWS_PER_WORKER
    pltpu.sync_copy(tables_hbm, tab_v)

    lane = lax.iota(jnp.int32, LANES)
    dvec = lane & 3
    grp_field = lane >> 2
    fvecs = [grp_field + 4 * g for g in range(3)]
    tvecs = [(grp_field + 4 * g) * TAB_SZ + dvec for g in range(3)]
    f3 = 12 + jnp.maximum(lane - 3, 0)
    t3 = 12 * TAB_SZ + dvec
    is_emb3 = lane < 4

    for p in range(2):
        pltpu.sync_copy(user_sparse_hbm.at[pl.ds(base + p * HALF, HALF)], in_v)

        def row_body(r, _p=p):
            rv = jnp.full((LANES,), r, jnp.int32)
            wbase = (_p * HALF + r) * OUT_DIM
            for g in range(3):
                col = plsc.load_gather(in_v, [rv, fvecs[g]])
                val = plsc.load_gather(tab_v, [col * EMB_DIM + tvecs[g]])
                wv = wbase + g * LANES + lane
                plsc.store_scatter(out_v, [wv >> 7, wv & (OUT_MINOR - 1)], val)
            col3 = plsc.load_gather(in_v, [rv, f3])
            emb3 = plsc.load_gather(tab_v, [col3 * EMB_DIM + t3])
            val3 = jnp.where(is_emb3, emb3, col3.astype(jnp.float32))
            wv = wbase + 3 * LANES + lane
            plsc.store_scatter(out_v, [wv >> 7, wv & (OUT_MINOR - 1)], val3)

        plsc.parallel_loop(0, HALF, 1, unroll=8)(row_body)

    pltpu.sync_copy(out_v,
                    out_hbm.at[pl.ds(wid * OROWS_PER_WORKER, OROWS_PER_WORKER)])


def kernel(user_sparse, tables):
    mesh = plsc.VectorSubcoreMesh(core_axis_name="c", subcore_axis_name="s")
    fn = pl.kernel(
        _sc_body,
        out_type=jax.ShapeDtypeStruct((OUT_MAJOR, OUT_MINOR), jnp.float32),
        mesh=mesh,
        compiler_params=pltpu.CompilerParams(needs_layout_passes=False),
        scratch_types=[
            pltpu.VMEM((HALF, N_FIELDS), jnp.int32),
            pltpu.VMEM((N_EMB * TAB_SZ,), jnp.float32),
            pltpu.VMEM((OROWS_PER_WORKER, OUT_MINOR), jnp.float32),
        ],
    )
    out_packed = fn(user_sparse, tables.reshape(-1))
    return out_packed.reshape(BATCH, OUT_DIM)


# speedup vs baseline: 1.0208x; 1.0208x over previous
import jax
import jax.numpy as jnp
from jax import lax
from jax.experimental import pallas as pl
from jax.experimental.pallas import tpu as pltpu
from jax.experimental.pallas import tpu_sc as plsc

BATCH = 16384
N_FIELDS = 26
N_EMB = 13
VOCAB = 500
EMB_DIM = 4
TAB_SZ = VOCAB * EMB_DIM
OUT_DIM = N_EMB * EMB_DIM + (N_FIELDS - 1 - N_EMB)
NUM_CORES = 2
NUM_SUBCORES = 16
NUM_WORKERS = NUM_CORES * NUM_SUBCORES
ROWS_PER_WORKER = BATCH // NUM_WORKERS
HALF = ROWS_PER_WORKER // 2
LANES = 16


def _sc_body(buf_hbm, out_hbm, in_v, tab_v, out_v):
    wid = lax.axis_index("s") * NUM_CORES + lax.axis_index("c")
    base = wid * ROWS_PER_WORKER
    pltpu.sync_copy(buf_hbm.at[pl.ds(BATCH * N_FIELDS, N_EMB * TAB_SZ)], tab_v)

    lane = lax.iota(jnp.int32, LANES)
    dvec = lane & 3
    grp_field = lane >> 2
    fvecs = [grp_field + 4 * g for g in range(3)]
    tvecs = [(grp_field + 4 * g) * TAB_SZ + dvec for g in range(3)]
    f3 = 12 + jnp.maximum(lane - 3, 0)
    t3 = 12 * TAB_SZ + dvec
    is_emb3 = lane < 4

    for p in range(2):
        pltpu.sync_copy(
            buf_hbm.at[pl.ds((base + p * HALF) * N_FIELDS, HALF * N_FIELDS)],
            in_v)

        def row_body(r, _p=p):
            rbase = r * N_FIELDS
            orv = jnp.full((LANES,), r + _p * HALF, jnp.int32)
            for g in range(3):
                col = plsc.load_gather(in_v, [rbase + fvecs[g]])
                vali = plsc.load_gather(tab_v, [col * EMB_DIM + tvecs[g]])
                plsc.store_scatter(out_v, [orv, g * LANES + lane],
                                   plsc.bitcast(vali, jnp.float32))
            col3 = plsc.load_gather(in_v, [rbase + f3])
            emb3i = plsc.load_gather(tab_v, [col3 * EMB_DIM + t3])
            val3 = jnp.where(is_emb3, plsc.bitcast(emb3i, jnp.float32),
                             col3.astype(jnp.float32))
            plsc.store_scatter(out_v, [orv, 3 * LANES + lane], val3)

        plsc.parallel_loop(0, HALF, 1, unroll=8)(row_body)

    pltpu.sync_copy(out_v, out_hbm.at[pl.ds(base, ROWS_PER_WORKER)])


def kernel(user_sparse, tables):
    mesh = plsc.VectorSubcoreMesh(core_axis_name="c", subcore_axis_name="s")
    fn = pl.kernel(
        _sc_body,
        out_type=jax.ShapeDtypeStruct((BATCH, OUT_DIM), jnp.float32),
        mesh=mesh,
        compiler_params=pltpu.CompilerParams(needs_layout_passes=False),
        scratch_types=[
            pltpu.VMEM((HALF * N_FIELDS,), jnp.int32),
            pltpu.VMEM((N_EMB * TAB_SZ,), jnp.int32),
            pltpu.VMEM((ROWS_PER_WORKER, OUT_DIM), jnp.float32),
        ],
    )
    tab_i = jax.lax.bitcast_convert_type(tables, jnp.int32).reshape(-1)
    buf = jnp.concatenate([user_sparse.reshape(-1), tab_i])
    return fn(buf)


# speedup vs baseline: 1.1409x; 1.1176x over previous
import jax
import jax.numpy as jnp
from jax import lax
from jax.experimental import pallas as pl
from jax.experimental.pallas import tpu as pltpu
from jax.experimental.pallas import tpu_sc as plsc

BATCH = 16384
N_FIELDS = 26
N_EMB = 13
VOCAB = 500
EMB_DIM = 4
TAB_SZ = VOCAB * EMB_DIM
OUT_DIM = N_EMB * EMB_DIM + (N_FIELDS - 1 - N_EMB)
NUM_CORES = 2
NUM_SUBCORES = 16
NUM_WORKERS = NUM_CORES * NUM_SUBCORES
ROWS_PER_WORKER = BATCH // NUM_WORKERS
HALF = ROWS_PER_WORKER // 2
LANES = 16


def _sc_body(user_sparse_hbm, tables_hbm, out_hbm, in_v, tab_v, out_v):
    wid = lax.axis_index("s") * NUM_CORES + lax.axis_index("c")
    base = wid * ROWS_PER_WORKER
    pltpu.sync_copy(tables_hbm, tab_v)

    lane = lax.iota(jnp.int32, LANES)
    dvec = lane & 3
    grp_field = lane >> 2
    fvecs = [grp_field + 4 * g for g in range(3)]
    tvecs = [(grp_field + 4 * g) * TAB_SZ + dvec for g in range(3)]
    f3 = 12 + jnp.maximum(lane - 3, 0)
    t3 = 12 * TAB_SZ + dvec
    is_emb3 = lane < 4

    for p in range(2):
        pltpu.sync_copy(user_sparse_hbm.at[pl.ds(base + p * HALF, HALF)], in_v)

        def row_body(r, _p=p):
            rv = jnp.full((LANES,), r, jnp.int32)
            orv = rv + _p * HALF
            for g in range(3):
                col = plsc.load_gather(in_v, [rv, fvecs[g]])
                val = plsc.load_gather(tab_v, [col * EMB_DIM + tvecs[g]])
                plsc.store_scatter(out_v, [orv, g * LANES + lane], val)
            col3 = plsc.load_gather(in_v, [rv, f3])
            emb3 = plsc.load_gather(tab_v, [col3 * EMB_DIM + t3])
            val3 = jnp.where(is_emb3, emb3, col3.astype(jnp.float32))
            plsc.store_scatter(out_v, [orv, 3 * LANES + lane], val3)

        plsc.parallel_loop(0, HALF, 1, unroll=16)(row_body)

    pltpu.sync_copy(out_v, out_hbm.at[pl.ds(base, ROWS_PER_WORKER)])


def kernel(user_sparse, tables):
    mesh = plsc.VectorSubcoreMesh(core_axis_name="c", subcore_axis_name="s")
    fn = pl.kernel(
        _sc_body,
        out_type=jax.ShapeDtypeStruct((BATCH, OUT_DIM), jnp.float32),
        mesh=mesh,
        compiler_params=pltpu.CompilerParams(needs_layout_passes=False),
        scratch_types=[
            pltpu.VMEM((HALF, N_FIELDS), jnp.int32),
            pltpu.VMEM((N_EMB * TAB_SZ,), jnp.float32),
            pltpu.VMEM((ROWS_PER_WORKER, OUT_DIM), jnp.float32),
        ],
    )
    return fn(user_sparse, tables.reshape(-1))


# speedup vs baseline: 1.1772x; 1.0318x over previous
import jax
import jax.numpy as jnp
from jax import lax
from jax.experimental import pallas as pl
from jax.experimental.pallas import tpu as pltpu
from jax.experimental.pallas import tpu_sc as plsc

BATCH = 16384
N_FIELDS = 26
N_EMB = 13
VOCAB = 500
EMB_DIM = 4
TAB_SZ = VOCAB * EMB_DIM
OUT_DIM = N_EMB * EMB_DIM + (N_FIELDS - 1 - N_EMB)
NUM_CORES = 2
NUM_SUBCORES = 16
NUM_WORKERS = NUM_CORES * NUM_SUBCORES
ROWS_PER_WORKER = BATCH // NUM_WORKERS
HALF = ROWS_PER_WORKER // 2
LANES = 16


def _sc_body(user_sparse_hbm, tables_hbm, out_hbm, in_v, tab_v, out_v):
    wid = lax.axis_index("s") * NUM_CORES + lax.axis_index("c")
    base = wid * ROWS_PER_WORKER
    pltpu.sync_copy(tables_hbm, tab_v)

    lane = lax.iota(jnp.int32, LANES)
    dvec = lane & 3
    grp_field = lane >> 2
    fvecs = [grp_field + 4 * g for g in range(3)]
    tvecs = [(grp_field + 4 * g) * TAB_SZ + dvec for g in range(3)]
    f3 = 12 + jnp.maximum(lane - 3, 0)
    t3 = 12 * TAB_SZ + dvec
    is_emb3 = lane < 4

    for p in range(2):
        pltpu.sync_copy(user_sparse_hbm.at[pl.ds(base + p * HALF, HALF)], in_v)

        def row_body(r, _p=p):
            rv = jnp.full((LANES,), r, jnp.int32)
            orv = rv + _p * HALF
            for g in range(3):
                col = plsc.load_gather(in_v, [rv, fvecs[g]])
                val = plsc.load_gather(tab_v, [col * EMB_DIM + tvecs[g]])
                plsc.store_scatter(out_v, [orv, g * LANES + lane], val)
            col3 = plsc.load_gather(in_v, [rv, f3])
            emb3 = plsc.load_gather(tab_v, [col3 * EMB_DIM + t3])
            val3 = jnp.where(is_emb3, emb3, col3.astype(jnp.float32))
            plsc.store_scatter(out_v, [orv, 3 * LANES + lane], val3)

        plsc.parallel_loop(0, HALF, 1, unroll=8)(row_body)

    pltpu.sync_copy(out_v, out_hbm.at[pl.ds(base, ROWS_PER_WORKER)])


def kernel(user_sparse, tables):
    mesh = plsc.VectorSubcoreMesh(core_axis_name="c", subcore_axis_name="s")
    fn = pl.kernel(
        _sc_body,
        out_type=jax.ShapeDtypeStruct((BATCH, OUT_DIM), jnp.float32),
        mesh=mesh,
        compiler_params=pltpu.CompilerParams(needs_layout_passes=False),
        scratch_types=[
            pltpu.VMEM((HALF, N_FIELDS), jnp.int32),
            pltpu.VMEM((N_EMB * TAB_SZ,), jnp.float32),
            pltpu.VMEM((ROWS_PER_WORKER, OUT_DIM), jnp.float32),
        ],
    )
    return fn(user_sparse, tables.reshape(-1))
